# baseline (speedup 1.0000x reference)
import functools

import jax
import jax.numpy as jnp
from jax import lax
from jax.experimental import pallas as pl
from jax.experimental.pallas import tpu as pltpu
from jax.experimental.pallas import tpu_sc as plsc

N_NODES = 10000
N_EDGES = 160000
D = 256
NUM_LABELS = 16

NC = 2
NS = 16
NW = NC * NS
SLAB = 320
PAD_N = SLAB * NW
ACC_ROWS = SLAB + 16
LANES = 16
EBLK = 2048
NBLK = -(-N_EDGES // EBLK)
E_PAD = NBLK * EBLK
FLUSH = 64
PEND = 192


@functools.cache
def _build_sc_segment_sum():
    mesh = plsc.VectorSubcoreMesh(
        core_axis_name="c", subcore_axis_name="s",
        num_cores=NC, num_subcores=NS)

    @functools.partial(
        pl.kernel,
        out_type=jax.ShapeDtypeStruct((PAD_N * D,), jnp.float32),
        mesh=mesh,
        compiler_params=pltpu.CompilerParams(needs_layout_passes=False),
        scratch_types=[
            pltpu.VMEM((EBLK,), jnp.int32),
            pltpu.VMEM((EBLK,), jnp.int32),
            pltpu.VMEM((PEND,), jnp.int32),
            pltpu.VMEM((PEND,), jnp.int32),
            pltpu.VMEM((FLUSH, D), jnp.float32),
            pltpu.VMEM((ACC_ROWS * D,), jnp.float32),
            pltpu.SemaphoreType.DMA,
        ],
    )
    def sc_segment_sum(h_hbm, src_hbm, dst_hbm, z_hbm, agg_hbm,
                       src_v, dst_v, psrc_v, prel_v, st_v, acc_v, sem):
        c = lax.axis_index("c")
        s = lax.axis_index("s")
        w = c * NS + s
        lo = w * SLAB
        cols = [lax.iota(jnp.int32, LANES) + k * LANES for k in range(D // LANES)]

        pltpu.sync_copy(z_hbm, acc_v)

        def flush(p):
            pltpu.async_copy(h_hbm.at[psrc_v.at[pl.ds(0, FLUSH)]], st_v,
                             sem).wait()

            def sub(u, _):
                for j in range(LANES):
                    row = plsc.load_gather(
                        prel_v, [jnp.full((LANES,), j, jnp.int32) + u * LANES])
                    base = row * D
                    for k in range(D // LANES):
                        vals = st_v[u * LANES + j, pl.ds(k * LANES, LANES)]
                        plsc.addupdate_scatter(acc_v, [base + cols[k]], vals)
                return _

            lax.fori_loop(0, FLUSH // LANES, sub, None)
            psrc_v[pl.ds(0, LANES)] = psrc_v[pl.ds(FLUSH, LANES)]
            prel_v[pl.ds(0, LANES)] = prel_v[pl.ds(FLUSH, LANES)]
            return p - FLUSH

        def scan_block(b, p):
            pltpu.sync_copy(src_hbm.at[pl.ds(b * EBLK, EBLK)], src_v)
            pltpu.sync_copy(dst_hbm.at[pl.ds(b * EBLK, EBLK)], dst_v)

            def group(g, p):
                rel = dst_v[pl.ds(g * LANES, LANES)] - lo
                m = (rel >= 0) & (rel < SLAB)
                plsc.store_compressed(psrc_v.at[pl.ds(p, LANES)],
                                      src_v[pl.ds(g * LANES, LANES)], mask=m)
                plsc.store_compressed(prel_v.at[pl.ds(p, LANES)], rel, mask=m)
                p = p + jnp.sum(m.astype(jnp.int32))
                return lax.while_loop(lambda q: q >= FLUSH, flush, p)

            return lax.fori_loop(0, EBLK // LANES, group, p)

        p = lax.fori_loop(0, NBLK, scan_block, jnp.int32(0))
        dummy = jnp.full((LANES,), SLAB, jnp.int32)
        zidx = jnp.zeros((LANES,), jnp.int32)
        for t in range(FLUSH // LANES):
            psrc_v[pl.ds(p + t * LANES, LANES)] = zidx
            prel_v[pl.ds(p + t * LANES, LANES)] = dummy
        flush(p)
        pltpu.sync_copy(acc_v.at[pl.ds(0, SLAB * D)],
                        agg_hbm.at[pl.ds(lo * D, SLAB * D)])

    return sc_segment_sum


_BN = 1000
_GRID = N_NODES // _BN


def _layer_body(scale_ref, h_ref, agg_ref, wg_ref, bg_ref, wr_ref, br_ref,
                o_ref):
    bf = jnp.bfloat16
    t = scale_ref[...] * h_ref[...] + agg_ref[...]
    z = jnp.dot(t.astype(bf), wg_ref[...].astype(bf),
                preferred_element_type=jnp.float32)
    h1 = jnp.maximum(z + bg_ref[...], 0.0)
    r = jnp.dot(h1.astype(bf), wr_ref[...].astype(bf),
                preferred_element_type=jnp.float32)
    o_ref[...] = jnp.maximum(r + br_ref[...], 0.0) + h1


_tc_layer = pl.pallas_call(
    _layer_body,
    grid=(_GRID,),
    in_specs=[
        pl.BlockSpec((1, D), lambda i: (0, 0)),
        pl.BlockSpec((_BN, D), lambda i: (i, 0)),
        pl.BlockSpec((_BN, D), lambda i: (i, 0)),
        pl.BlockSpec((D, D), lambda i: (0, 0)),
        pl.BlockSpec((1, D), lambda i: (0, 0)),
        pl.BlockSpec((D, D), lambda i: (0, 0)),
        pl.BlockSpec((1, D), lambda i: (0, 0)),
    ],
    out_specs=pl.BlockSpec((_BN, D), lambda i: (i, 0)),
    out_shape=jax.ShapeDtypeStruct((N_NODES, D), jnp.float32),
)


def _att_body(h0_ref, h1_ref, wa_ref, wo_ref, bo_ref, o_ref):
    bf = jnp.bfloat16
    h0 = h0_ref[...]
    h1 = h1_ref[...]
    wa = wa_ref[...].astype(bf)
    wo = wo_ref[...]
    dn = (((1,), (1,)), ((), ()))
    s0 = lax.dot_general(h0.astype(bf), wa, dn,
                         preferred_element_type=jnp.float32)
    s1 = lax.dot_general(h1.astype(bf), wa, dn,
                         preferred_element_type=jnp.float32)
    p0 = lax.dot_general(h0, wo, dn, preferred_element_type=jnp.float32,
                         precision=lax.Precision.HIGHEST)
    p1 = lax.dot_general(h1, wo, dn, preferred_element_type=jnp.float32,
                         precision=lax.Precision.HIGHEST)
    m = jnp.maximum(s0, s1)
    e0 = jnp.exp(s0 - m)
    e1 = jnp.exp(s1 - m)
    o_ref[...] = (e0 * p0 + e1 * p1) / (e0 + e1) + bo_ref[...]


_tc_att = pl.pallas_call(
    _att_body,
    grid=(_GRID,),
    in_specs=[
        pl.BlockSpec((_BN, D), lambda i: (i, 0)),
        pl.BlockSpec((_BN, D), lambda i: (i, 0)),
        pl.BlockSpec((NUM_LABELS, D), lambda i: (0, 0)),
        pl.BlockSpec((1, D), lambda i: (0, 0)),
        pl.BlockSpec((1, NUM_LABELS), lambda i: (0, 0)),
    ],
    out_specs=pl.BlockSpec((_BN, NUM_LABELS), lambda i: (i, 0)),
    out_shape=jax.ShapeDtypeStruct((N_NODES, NUM_LABELS), jnp.float32),
)


def kernel(x, edge_index, W_gin0, b_gin0, eps0, W_res0, b_res0,
           W_gin1, b_gin1, eps1, W_res1, b_res1, W_att, W_out, b_out):
    pad = E_PAD - N_EDGES
    src = jnp.concatenate([edge_index[0], jnp.zeros((pad,), jnp.int32)])
    dst = jnp.concatenate([edge_index[1], jnp.full((pad,), -1, jnp.int32)])
    zeros = jnp.zeros((ACC_ROWS * D,), jnp.float32)

    sc_segment_sum = _build_sc_segment_sum()
    agg0 = sc_segment_sum(x, src, dst, zeros).reshape(PAD_N, D)
    h0 = _tc_layer((1.0 + eps0).reshape(1, 1) * jnp.ones((1, D), jnp.float32),
                   x, agg0, W_gin0, b_gin0.reshape(1, D),
                   W_res0, b_res0.reshape(1, D))
    agg1 = sc_segment_sum(h0, src, dst, zeros).reshape(PAD_N, D)
    h1 = _tc_layer((1.0 + eps1).reshape(1, 1) * jnp.ones((1, D), jnp.float32),
                   h0, agg1, W_gin1, b_gin1.reshape(1, D),
                   W_res1, b_res1.reshape(1, D))
    return _tc_att(h0, h1, W_att, W_out.reshape(1, D),
                   jnp.broadcast_to(b_out.reshape(1, 1), (1, NUM_LABELS)))


# speedup vs baseline: 1.0973x; 1.0973x over previous
import functools

import jax
import jax.numpy as jnp
from jax import lax
from jax.experimental import pallas as pl
from jax.experimental.pallas import tpu as pltpu
from jax.experimental.pallas import tpu_sc as plsc

N_NODES = 10000
N_EDGES = 160000
D = 256
NUM_LABELS = 16

NC = 2
NS = 16
NW = NC * NS
SLAB = 320
PAD_N = SLAB * NW
ACC_ROWS = SLAB + 16
LANES = 16
EBLK = 2048
NBLK = -(-N_EDGES // EBLK)
E_PAD = NBLK * EBLK
FLUSH = 64
PEND = 192


@functools.cache
def _build_sc_segment_sum():
    mesh = plsc.VectorSubcoreMesh(
        core_axis_name="c", subcore_axis_name="s",
        num_cores=NC, num_subcores=NS)

    @functools.partial(
        pl.kernel,
        out_type=jax.ShapeDtypeStruct((PAD_N * D,), jnp.float32),
        mesh=mesh,
        compiler_params=pltpu.CompilerParams(needs_layout_passes=False),
        scratch_types=[
            pltpu.VMEM((2, EBLK), jnp.int32),
            pltpu.VMEM((2, EBLK), jnp.int32),
            pltpu.VMEM((PEND,), jnp.int32),
            pltpu.VMEM((PEND,), jnp.int32),
            pltpu.VMEM((FLUSH, D), jnp.float32),
            pltpu.VMEM((ACC_ROWS * D,), jnp.float32),
            pltpu.SemaphoreType.DMA,
            pltpu.SemaphoreType.DMA,
            pltpu.SemaphoreType.DMA,
        ],
    )
    def sc_segment_sum(h_hbm, src_hbm, dst_hbm, z_hbm, agg_hbm,
                       src_v, dst_v, psrc_v, prel_v, st_v, acc_v,
                       sem, bsem_s, bsem_d):
        c = lax.axis_index("c")
        s = lax.axis_index("s")
        w = c * NS + s
        lo = w * SLAB
        cols = [lax.iota(jnp.int32, LANES) + k * LANES for k in range(D // LANES)]

        pltpu.sync_copy(z_hbm, acc_v)

        def flush(p):
            pltpu.async_copy(h_hbm.at[psrc_v.at[pl.ds(0, FLUSH)]], st_v,
                             sem).wait()

            def sub(u, _):
                for j in range(LANES):
                    row = plsc.load_gather(
                        prel_v, [jnp.full((LANES,), j, jnp.int32) + u * LANES])
                    base = row * D
                    for k in range(D // LANES):
                        vals = st_v[u * LANES + j, pl.ds(k * LANES, LANES)]
                        plsc.addupdate_scatter(acc_v, [base + cols[k]], vals)
                return _

            lax.fori_loop(0, FLUSH // LANES, sub, None)
            psrc_v[pl.ds(0, LANES)] = psrc_v[pl.ds(FLUSH, LANES)]
            prel_v[pl.ds(0, LANES)] = prel_v[pl.ds(FLUSH, LANES)]
            return p - FLUSH

        def start_blk(b, slot):
            pltpu.async_copy(src_hbm.at[pl.ds(b * EBLK, EBLK)],
                             src_v.at[slot], bsem_s)
            pltpu.async_copy(dst_hbm.at[pl.ds(b * EBLK, EBLK)],
                             dst_v.at[slot], bsem_d)

        def scan_block(b, p):
            slot = b % 2
            pltpu.make_async_copy(src_hbm.at[pl.ds(b * EBLK, EBLK)],
                                  src_v.at[slot], bsem_s).wait()
            pltpu.make_async_copy(dst_hbm.at[pl.ds(b * EBLK, EBLK)],
                                  dst_v.at[slot], bsem_d).wait()

            @pl.when(b + 1 < NBLK)
            def _():
                start_blk(b + 1, (b + 1) % 2)

            def group(g, p):
                rel = dst_v[slot, pl.ds(g * LANES, LANES)] - lo
                m = (rel >= 0) & (rel < SLAB)
                plsc.store_compressed(psrc_v.at[pl.ds(p, LANES)],
                                      src_v[slot, pl.ds(g * LANES, LANES)],
                                      mask=m)
                plsc.store_compressed(prel_v.at[pl.ds(p, LANES)], rel, mask=m)
                p = p + jnp.sum(m.astype(jnp.int32))
                return lax.while_loop(lambda q: q >= FLUSH, flush, p)

            return lax.fori_loop(0, EBLK // LANES, group, p)

        start_blk(0, 0)
        p = lax.fori_loop(0, NBLK, scan_block, jnp.int32(0))
        dummy = jnp.full((LANES,), SLAB, jnp.int32)
        zidx = jnp.zeros((LANES,), jnp.int32)
        for t in range(FLUSH // LANES):
            psrc_v[pl.ds(p + t * LANES, LANES)] = zidx
            prel_v[pl.ds(p + t * LANES, LANES)] = dummy
        flush(p)
        pltpu.sync_copy(acc_v.at[pl.ds(0, SLAB * D)],
                        agg_hbm.at[pl.ds(lo * D, SLAB * D)])

    return sc_segment_sum


_BN = 1000
_GRID = N_NODES // _BN


def _layer_body(scale_ref, h_ref, agg_ref, wg_ref, bg_ref, wr_ref, br_ref,
                o_ref):
    bf = jnp.bfloat16
    t = scale_ref[...] * h_ref[...] + agg_ref[...]
    z = jnp.dot(t.astype(bf), wg_ref[...].astype(bf),
                preferred_element_type=jnp.float32)
    h1 = jnp.maximum(z + bg_ref[...], 0.0)
    r = jnp.dot(h1.astype(bf), wr_ref[...].astype(bf),
                preferred_element_type=jnp.float32)
    o_ref[...] = jnp.maximum(r + br_ref[...], 0.0) + h1


_tc_layer = pl.pallas_call(
    _layer_body,
    grid=(_GRID,),
    in_specs=[
        pl.BlockSpec((1, D), lambda i: (0, 0)),
        pl.BlockSpec((_BN, D), lambda i: (i, 0)),
        pl.BlockSpec((_BN, D), lambda i: (i, 0)),
        pl.BlockSpec((D, D), lambda i: (0, 0)),
        pl.BlockSpec((1, D), lambda i: (0, 0)),
        pl.BlockSpec((D, D), lambda i: (0, 0)),
        pl.BlockSpec((1, D), lambda i: (0, 0)),
    ],
    out_specs=pl.BlockSpec((_BN, D), lambda i: (i, 0)),
    out_shape=jax.ShapeDtypeStruct((N_NODES, D), jnp.float32),
)


def _att_body(h0_ref, h1_ref, wa_ref, wo_ref, bo_ref, o_ref):
    bf = jnp.bfloat16
    h0 = h0_ref[...]
    h1 = h1_ref[...]
    wa = wa_ref[...].astype(bf)
    wo = wo_ref[...]
    dn = (((1,), (1,)), ((), ()))
    s0 = lax.dot_general(h0.astype(bf), wa, dn,
                         preferred_element_type=jnp.float32)
    s1 = lax.dot_general(h1.astype(bf), wa, dn,
                         preferred_element_type=jnp.float32)
    p0 = lax.dot_general(h0, wo, dn, preferred_element_type=jnp.float32,
                         precision=lax.Precision.HIGHEST)
    p1 = lax.dot_general(h1, wo, dn, preferred_element_type=jnp.float32,
                         precision=lax.Precision.HIGHEST)
    m = jnp.maximum(s0, s1)
    e0 = jnp.exp(s0 - m)
    e1 = jnp.exp(s1 - m)
    o_ref[...] = (e0 * p0 + e1 * p1) / (e0 + e1) + bo_ref[...]


_tc_att = pl.pallas_call(
    _att_body,
    grid=(_GRID,),
    in_specs=[
        pl.BlockSpec((_BN, D), lambda i: (i, 0)),
        pl.BlockSpec((_BN, D), lambda i: (i, 0)),
        pl.BlockSpec((NUM_LABELS, D), lambda i: (0, 0)),
        pl.BlockSpec((1, D), lambda i: (0, 0)),
        pl.BlockSpec((1, NUM_LABELS), lambda i: (0, 0)),
    ],
    out_specs=pl.BlockSpec((_BN, NUM_LABELS), lambda i: (i, 0)),
    out_shape=jax.ShapeDtypeStruct((N_NODES, NUM_LABELS), jnp.float32),
)


def kernel(x, edge_index, W_gin0, b_gin0, eps0, W_res0, b_res0,
           W_gin1, b_gin1, eps1, W_res1, b_res1, W_att, W_out, b_out):
    pad = E_PAD - N_EDGES
    src = jnp.concatenate([edge_index[0], jnp.zeros((pad,), jnp.int32)])
    dst = jnp.concatenate([edge_index[1], jnp.full((pad,), -1, jnp.int32)])
    zeros = jnp.zeros((ACC_ROWS * D,), jnp.float32)

    sc_segment_sum = _build_sc_segment_sum()
    agg0 = sc_segment_sum(x, src, dst, zeros).reshape(PAD_N, D)
    h0 = _tc_layer((1.0 + eps0).reshape(1, 1) * jnp.ones((1, D), jnp.float32),
                   x, agg0, W_gin0, b_gin0.reshape(1, D),
                   W_res0, b_res0.reshape(1, D))
    agg1 = sc_segment_sum(h0, src, dst, zeros).reshape(PAD_N, D)
    h1 = _tc_layer((1.0 + eps1).reshape(1, 1) * jnp.ones((1, D), jnp.float32),
                   h0, agg1, W_gin1, b_gin1.reshape(1, D),
                   W_res1, b_res1.reshape(1, D))
    return _tc_att(h0, h1, W_att, W_out.reshape(1, D),
                   jnp.broadcast_to(b_out.reshape(1, 1), (1, NUM_LABELS)))
